# baseline (speedup 1.0000x reference)
import functools

import jax
import jax.numpy as jnp
from jax import lax
from jax.experimental import pallas as pl
from jax.experimental.pallas import tpu as pltpu
from jax.experimental.pallas import tpu_sc as plsc

N = 10000
E = 320000
D = 128
DE = 16
G = 64
OUT = 64

NC = 2
NS = 16
NW = NC * NS

EB = 64
IC = 32
BPW0 = 160
BPW1 = 160
NB0 = NS * BPW0
E_PAD = NS * (BPW0 + BPW1) * EB
ACC_ROWS = 10240
RPT = ACC_ROWS // NS

_sc_mesh = plsc.VectorSubcoreMesh(core_axis_name="c", subcore_axis_name="s")


@functools.partial(
    pl.kernel,
    out_type=jax.ShapeDtypeStruct((NC, ACC_ROWS, D), jnp.float32),
    mesh=_sc_mesh,
    scratch_types=[
        pltpu.VMEM((IC, EB), jnp.int32),
        pltpu.VMEM((IC, EB), jnp.int32),
        pltpu.VMEM((EB, D), jnp.float32),
        pltpu.VMEM((EB, D), jnp.float32),
        pltpu.VMEM((EB, D), jnp.float32),
        pltpu.VMEM((EB, D), jnp.float32),
        pltpu.VMEM_SHARED((ACC_ROWS, D), jnp.float32),
        pltpu.SemaphoreType.DMA,
        pltpu.SemaphoreType.DMA,
        pltpu.SemaphoreType.DMA,
        pltpu.SemaphoreType.DMA,
    ],
)
def _sc_aggregate(h_hbm, el_hbm, er_hbm, src_hbm, dst_hbm, out_hbm,
                  src_v, dst_v, rows0_v, rows1_v, ev0_v, ev1_v,
                  acc_sh, gsem0, gsem1, esem0, esem1):
    c = lax.axis_index("c")
    s = lax.axis_index("s")
    bpw = jnp.where(c == 0, BPW0, BPW1)
    wbase = jnp.where(c == 0, s * BPW0, NB0 + s * BPW1)

    rows_b = (rows0_v, rows1_v)
    ev_b = (ev0_v, ev1_v)
    gsem_b = (gsem0, gsem1)
    esem_b = (esem0, esem1)

    def zrow(r, carry):
        for kk in range(D // 16):
            rows0_v[r, pl.ds(kk * 16, 16)] = jnp.zeros((16,), jnp.float32)
        return carry

    lax.fori_loop(0, EB, zrow, 0)

    def zcopy(r, carry):
        pltpu.sync_copy(rows0_v, acc_sh.at[pl.ds(s * RPT + r * EB, EB)])
        return carry

    lax.fori_loop(0, RPT // EB, zcopy, 0)
    plsc.subcore_barrier()

    def chunk(cc, carry):
        base = wbase + cc * IC
        pltpu.sync_copy(src_hbm.at[pl.ds(base, IC)], src_v)
        pltpu.sync_copy(dst_hbm.at[pl.ds(base, IC)], dst_v)

        e_arr = (el_hbm, er_hbm)

        def issue(jj, b):
            pltpu.async_copy(h_hbm.at[src_v.at[jj]], rows_b[b], gsem_b[b])
            pltpu.async_copy(
                e_arr[b].at[pl.ds((base + jj) // 2 * EB, EB)],
                ev_b[b], esem_b[b])

        issue(0, 0)

        def pair(p, carry1):
            for b in range(2):
                j = p * 2 + b
                nb = 1 - b

                @pl.when(j + 1 < IC)
                def _():
                    issue(j + 1, nb)

                pltpu.make_async_copy(
                    h_hbm.at[src_v.at[j]], rows_b[b], gsem_b[b]).wait()
                pltpu.make_async_copy(
                    e_arr[b].at[pl.ds((base + j) // 2 * EB, EB)],
                    ev_b[b], esem_b[b]).wait()

                rv, ev = rows_b[b], ev_b[b]

                def elem(r, carry2):
                    for kk in range(D // 16):
                        sl = pl.ds(kk * 16, 16)
                        v = rv[r, sl] + ev[r, sl]
                        rv[r, sl] = jnp.maximum(v, 0.0)
                    return carry2

                lax.fori_loop(0, EB, elem, 0)
                pltpu.sync_copy(rv, acc_sh.at[dst_v.at[j]], add=True)
            return carry1

        lax.fori_loop(0, IC // 2, pair, 0)
        return carry

    lax.fori_loop(0, bpw // IC, chunk, 0)
    plsc.subcore_barrier()
    pltpu.sync_copy(acc_sh.at[pl.ds(s * RPT, RPT)],
                    out_hbm.at[c, pl.ds(s * RPT, RPT)])


_E_RB = 1280


def _split_eo(e):
    e3 = e.reshape(_E_RB // (2 * EB), 2 * EB, D)
    ev = e3[:, :EB, :].reshape(_E_RB // 2, D)
    od = e3[:, EB:, :].reshape(_E_RB // 2, D)
    return ev, od


def _edge_mlp1_body(ea_ref, We_ref, be_ref, el_ref, er_ref):
    e = (jnp.dot(ea_ref[...], We_ref[...],
                 preferred_element_type=jnp.float32) + be_ref[...])
    el_ref[...], er_ref[...] = _split_eo(e)


def _edge_mlp2_body(ea_ref, We1_ref, be1_ref, We2_ref, be2_ref,
                    e1l_ref, e1r_ref, e2l_ref, e2r_ref):
    a = ea_ref[...]
    e1 = jnp.dot(a, We1_ref[...], preferred_element_type=jnp.float32) + be1_ref[...]
    e1l_ref[...], e1r_ref[...] = _split_eo(e1)
    e2 = jnp.dot(a, We2_ref[...], preferred_element_type=jnp.float32) + be2_ref[...]
    e2l_ref[...], e2r_ref[...] = _split_eo(e2)


_E_WSPEC = pl.BlockSpec((DE, D), lambda i: (0, 0))
_E_BSPEC = pl.BlockSpec((1, D), lambda i: (0, 0))
_E_ASPEC = pl.BlockSpec((_E_RB, DE), lambda i: (i, 0))
_E_OSPEC = pl.BlockSpec((_E_RB // 2, D), lambda i: (i, 0))
_E_OSHAPE = jax.ShapeDtypeStruct((E_PAD // 2, D), jnp.float32)


def _edge_mlp1(ea, We, be):
    return pl.pallas_call(
        _edge_mlp1_body,
        grid=(E_PAD // _E_RB,),
        in_specs=[_E_ASPEC, _E_WSPEC, _E_BSPEC],
        out_specs=[_E_OSPEC, _E_OSPEC],
        out_shape=[_E_OSHAPE, _E_OSHAPE],
    )(ea, We, be)


def _edge_mlp2(ea, We1, be1, We2, be2):
    return pl.pallas_call(
        _edge_mlp2_body,
        grid=(E_PAD // _E_RB,),
        in_specs=[_E_ASPEC, _E_WSPEC, _E_BSPEC, _E_WSPEC, _E_BSPEC],
        out_specs=[_E_OSPEC] * 4,
        out_shape=[_E_OSHAPE] * 4,
    )(ea, We1, be1, We2, be2)


_N_RB = 400


def _update_body(h_ref, p0_ref, p1_ref, Wn_ref, bn_ref, o_ref):
    hs = h_ref[...] + p0_ref[0] + p1_ref[0]
    o_ref[...] = jnp.maximum(
        jnp.dot(hs, Wn_ref[...], preferred_element_type=jnp.float32) + bn_ref[...],
        0.0)


def _update(h, parts, Wn, bn):
    grid = (N // _N_RB,)
    return pl.pallas_call(
        _update_body,
        grid=grid,
        in_specs=[
            pl.BlockSpec((_N_RB, D), lambda i: (i, 0)),
            pl.BlockSpec((1, _N_RB, D), lambda i: (0, i, 0)),
            pl.BlockSpec((1, _N_RB, D), lambda i: (1, i, 0)),
            pl.BlockSpec((D, D), lambda i: (0, 0)),
            pl.BlockSpec((1, D), lambda i: (0, 0)),
        ],
        out_specs=pl.BlockSpec((_N_RB, D), lambda i: (i, 0)),
        out_shape=jax.ShapeDtypeStruct((N, D), jnp.float32),
    )(h, parts, parts, Wn, bn)


def _pool_body(h_ref, brow_ref, bcol_ref, W1_ref, b1_ref, W2_ref, b2_ref,
               o_ref, maxs_ref):
    h = h_ref[...]
    brow = brow_ref[...]
    gid = lax.broadcasted_iota(jnp.int32, (G, N), 0)
    onehot = (brow == gid).astype(jnp.float32)
    counts = jnp.sum(onehot, axis=1, keepdims=True)
    sums = jnp.dot(onehot, h, preferred_element_type=jnp.float32)
    mean = sums / jnp.maximum(counts, 1.0)

    bcol = bcol_ref[...]

    def gmax(g, carry):
        m = jnp.max(jnp.where(bcol == g, h, -1e30), axis=0, keepdims=True)
        maxs_ref[pl.ds(g, 1), :] = m
        return carry

    lax.fori_loop(0, G, gmax, 0)
    maxs = jnp.where(counts > 0, maxs_ref[...], 0.0)

    gf = jnp.concatenate([maxs, mean], axis=1)
    hid = jnp.maximum(
        jnp.dot(gf, W1_ref[...], preferred_element_type=jnp.float32) + b1_ref[...],
        0.0)
    o_ref[...] = jnp.dot(hid, W2_ref[...], preferred_element_type=jnp.float32) + b2_ref[...]


def _pool(h, brow, bcol, W1, b1, W2, b2):
    return pl.pallas_call(
        _pool_body,
        out_shape=jax.ShapeDtypeStruct((G, OUT), jnp.float32),
        scratch_shapes=[pltpu.VMEM((G, D), jnp.float32)],
    )(h, brow, bcol, W1, b1, W2, b2)


def kernel(x, edge_index, edge_attr, batch,
           We0, be0, Wn0, bn0, We1, be1, Wn1, bn1, We2, be2, Wn2, bn2,
           W1, b1, W2, b2):
    pad = E_PAD - E
    src = jnp.concatenate(
        [edge_index[0], jnp.arange(pad, dtype=jnp.int32) % N])
    dst = jnp.concatenate(
        [edge_index[1], N + (jnp.arange(pad, dtype=jnp.int32) % (ACC_ROWS - N))])
    srcp = src.reshape(E_PAD // EB, EB)
    dstp = dst.reshape(E_PAD // EB, EB)

    e0l, e0r = _edge_mlp1(edge_attr, We0, be0.reshape(1, D))
    parts = _sc_aggregate(x, e0l, e0r, srcp, dstp)
    e1l, e1r, e2l, e2r = _edge_mlp2(edge_attr, We1, be1.reshape(1, D),
                                    We2, be2.reshape(1, D))
    h = _update(x, parts, Wn0, bn0.reshape(1, D))

    for el, er, Wn, bn in ((e1l, e1r, Wn1, bn1), (e2l, e2r, Wn2, bn2)):
        parts = _sc_aggregate(h, el, er, srcp, dstp)
        h = _update(h, parts, Wn, bn.reshape(1, D))

    return _pool(h, batch.reshape(1, N), batch.reshape(N, 1),
                 W1, b1.reshape(1, D // 2), W2, b2.reshape(1, OUT))


# speedup vs baseline: 1.0795x; 1.0795x over previous
import functools

import jax
import jax.numpy as jnp
from jax import lax
from jax.experimental import pallas as pl
from jax.experimental.pallas import tpu as pltpu
from jax.experimental.pallas import tpu_sc as plsc

N = 10000
E = 320000
D = 128
DE = 16
G = 64
OUT = 64

NC = 2
NS = 16
NW = NC * NS

EB = 64
IC = 32
BPW0 = 160
BPW1 = 160
NB0 = NS * BPW0
E_PAD = NS * (BPW0 + BPW1) * EB
ACC_ROWS = 10240
RPT = ACC_ROWS // NS

_sc_mesh = plsc.VectorSubcoreMesh(core_axis_name="c", subcore_axis_name="s")


@functools.partial(
    pl.kernel,
    out_type=jax.ShapeDtypeStruct((NC, ACC_ROWS, D), jnp.float32),
    mesh=_sc_mesh,
    scratch_types=[
        pltpu.VMEM((IC, EB), jnp.int32),
        pltpu.VMEM((IC, EB), jnp.int32),
        pltpu.VMEM((EB, D), jnp.float32),
        pltpu.VMEM((EB, D), jnp.float32),
        pltpu.VMEM((EB, D), jnp.float32),
        pltpu.VMEM((EB, D), jnp.float32),
        pltpu.VMEM_SHARED((ACC_ROWS, D), jnp.float32),
        pltpu.SemaphoreType.DMA,
        pltpu.SemaphoreType.DMA,
        pltpu.SemaphoreType.DMA,
        pltpu.SemaphoreType.DMA,
    ],
)
def _sc_aggregate(h_hbm, el_hbm, er_hbm, src_hbm, dst_hbm, out_hbm,
                  src_v, dst_v, rows0_v, rows1_v, ev0_v, ev1_v,
                  acc_sh, gsem0, gsem1, esem0, esem1):
    c = lax.axis_index("c")
    s = lax.axis_index("s")
    bpw = jnp.where(c == 0, BPW0, BPW1)
    wbase = jnp.where(c == 0, s * BPW0, NB0 + s * BPW1)

    rows_b = (rows0_v, rows1_v)
    ev_b = (ev0_v, ev1_v)
    gsem_b = (gsem0, gsem1)
    esem_b = (esem0, esem1)

    def zrow(r, carry):
        for kk in range(D // 16):
            rows0_v[r, pl.ds(kk * 16, 16)] = jnp.zeros((16,), jnp.float32)
        return carry

    lax.fori_loop(0, EB, zrow, 0)

    def zcopy(r, carry):
        pltpu.sync_copy(rows0_v, acc_sh.at[pl.ds(s * RPT + r * EB, EB)])
        return carry

    lax.fori_loop(0, RPT // EB, zcopy, 0)
    plsc.subcore_barrier()

    def chunk(cc, carry):
        base = wbase + cc * IC
        pltpu.sync_copy(src_hbm.at[pl.ds(base, IC)], src_v)
        pltpu.sync_copy(dst_hbm.at[pl.ds(base, IC)], dst_v)

        e_arr = (el_hbm, er_hbm)

        def issue(jj, b):
            pltpu.async_copy(h_hbm.at[src_v.at[jj]], rows_b[b], gsem_b[b])
            pltpu.async_copy(
                e_arr[b].at[pl.ds((base + jj) // 2 * EB, EB)],
                ev_b[b], esem_b[b])

        issue(0, 0)

        def pair(p, carry1):
            for b in range(2):
                j = p * 2 + b
                nb = 1 - b

                @pl.when(j + 1 < IC)
                def _():
                    issue(j + 1, nb)

                pltpu.make_async_copy(
                    h_hbm.at[src_v.at[j]], rows_b[b], gsem_b[b]).wait()
                pltpu.make_async_copy(
                    e_arr[b].at[pl.ds((base + j) // 2 * EB, EB)],
                    ev_b[b], esem_b[b]).wait()

                rv, ev = rows_b[b], ev_b[b]

                def elem(r, carry2):
                    for kk in range(D // 16):
                        sl = pl.ds(kk * 16, 16)
                        v = rv[r, sl] + ev[r, sl]
                        rv[r, sl] = jnp.maximum(v, 0.0)
                    return carry2

                lax.fori_loop(0, EB, elem, 0)
                pltpu.sync_copy(rv, acc_sh.at[dst_v.at[j]], add=True)
            return carry1

        lax.fori_loop(0, IC // 2, pair, 0)
        return carry

    lax.fori_loop(0, bpw // IC, chunk, 0)
    plsc.subcore_barrier()
    pltpu.sync_copy(acc_sh.at[pl.ds(s * RPT, RPT)],
                    out_hbm.at[c, pl.ds(s * RPT, RPT)])


_E_RB = 1280


def _split_eo(e):
    e3 = e.reshape(_E_RB // (2 * EB), 2 * EB, D)
    ev = e3[:, :EB, :].reshape(_E_RB // 2, D)
    od = e3[:, EB:, :].reshape(_E_RB // 2, D)
    return ev, od


def _edge_mlp1_body(ea_ref, We_ref, be_ref, el_ref, er_ref):
    e = (jnp.dot(ea_ref[...], We_ref[...],
                 preferred_element_type=jnp.float32) + be_ref[...])
    el_ref[...], er_ref[...] = _split_eo(e)


def _edge_mlp2_body(ea_ref, We1_ref, be1_ref, We2_ref, be2_ref,
                    e1l_ref, e1r_ref, e2l_ref, e2r_ref):
    a = ea_ref[...]
    e1 = jnp.dot(a, We1_ref[...], preferred_element_type=jnp.float32) + be1_ref[...]
    e1l_ref[...], e1r_ref[...] = _split_eo(e1)
    e2 = jnp.dot(a, We2_ref[...], preferred_element_type=jnp.float32) + be2_ref[...]
    e2l_ref[...], e2r_ref[...] = _split_eo(e2)


_E_WSPEC = pl.BlockSpec((DE, D), lambda i: (0, 0))
_E_BSPEC = pl.BlockSpec((1, D), lambda i: (0, 0))
_E_ASPEC = pl.BlockSpec((_E_RB, DE), lambda i: (i, 0))
_E_OSPEC = pl.BlockSpec((_E_RB // 2, D), lambda i: (i, 0))
_E_OSHAPE = jax.ShapeDtypeStruct((E_PAD // 2, D), jnp.float32)


def _edge_mlp1(ea, We, be):
    return pl.pallas_call(
        _edge_mlp1_body,
        grid=(E_PAD // _E_RB,),
        in_specs=[_E_ASPEC, _E_WSPEC, _E_BSPEC],
        out_specs=[_E_OSPEC, _E_OSPEC],
        out_shape=[_E_OSHAPE, _E_OSHAPE],
    )(ea, We, be)


def _edge_mlp2(ea, We1, be1, We2, be2):
    return pl.pallas_call(
        _edge_mlp2_body,
        grid=(E_PAD // _E_RB,),
        in_specs=[_E_ASPEC, _E_WSPEC, _E_BSPEC, _E_WSPEC, _E_BSPEC],
        out_specs=[_E_OSPEC] * 4,
        out_shape=[_E_OSHAPE] * 4,
    )(ea, We1, be1, We2, be2)


_N_RB = 400


def _update_body(h_ref, p0_ref, p1_ref, Wn_ref, bn_ref, o_ref):
    hs = h_ref[...] + p0_ref[0] + p1_ref[0]
    o_ref[...] = jnp.maximum(
        jnp.dot(hs, Wn_ref[...], preferred_element_type=jnp.float32) + bn_ref[...],
        0.0)


def _update(h, parts, Wn, bn):
    grid = (N // _N_RB,)
    return pl.pallas_call(
        _update_body,
        grid=grid,
        in_specs=[
            pl.BlockSpec((_N_RB, D), lambda i: (i, 0)),
            pl.BlockSpec((1, _N_RB, D), lambda i: (0, i, 0)),
            pl.BlockSpec((1, _N_RB, D), lambda i: (1, i, 0)),
            pl.BlockSpec((D, D), lambda i: (0, 0)),
            pl.BlockSpec((1, D), lambda i: (0, 0)),
        ],
        out_specs=pl.BlockSpec((_N_RB, D), lambda i: (i, 0)),
        out_shape=jax.ShapeDtypeStruct((N, D), jnp.float32),
    )(h, parts, parts, Wn, bn)


_P_RB = 400
_P_NB = N // _P_RB


def _upool_body(h_ref, p0_ref, p1_ref, Wn_ref, bn_ref, bcol_ref,
                W1_ref, b1_ref, W2_ref, b2_ref, o_ref,
                maxs_ref, sums_ref, cnt_ref):
    i = pl.program_id(0)

    @pl.when(i == 0)
    def _():
        maxs_ref[...] = jnp.full((G, D), -1e30, jnp.float32)
        sums_ref[...] = jnp.zeros((G, D), jnp.float32)
        cnt_ref[...] = jnp.zeros((G, 1), jnp.float32)

    hs = h_ref[...] + p0_ref[0] + p1_ref[0]
    h3 = jnp.maximum(
        jnp.dot(hs, Wn_ref[...], preferred_element_type=jnp.float32) + bn_ref[...],
        0.0)

    bcol = bcol_ref[...]
    gmin = jnp.min(bcol)
    gmax = jnp.max(bcol)

    def gbody(g, carry):
        mask = bcol == g
        m = jnp.max(jnp.where(mask, h3, -1e30), axis=0, keepdims=True)
        maxs_ref[pl.ds(g, 1), :] = jnp.maximum(maxs_ref[pl.ds(g, 1), :], m)
        s = jnp.sum(jnp.where(mask, h3, 0.0), axis=0, keepdims=True)
        sums_ref[pl.ds(g, 1), :] += s
        cnt_ref[pl.ds(g, 1), :] += jnp.sum(
            mask.astype(jnp.float32), axis=0, keepdims=True)
        return carry

    lax.fori_loop(gmin, gmax + 1, gbody, 0)

    @pl.when(i == _P_NB - 1)
    def _():
        counts = cnt_ref[...]
        mean = sums_ref[...] / jnp.maximum(counts, 1.0)
        maxs = jnp.where(counts > 0, maxs_ref[...], 0.0)
        gf = jnp.concatenate([maxs, mean], axis=1)
        hid = jnp.maximum(
            jnp.dot(gf, W1_ref[...], preferred_element_type=jnp.float32) + b1_ref[...],
            0.0)
        o_ref[...] = jnp.dot(hid, W2_ref[...],
                             preferred_element_type=jnp.float32) + b2_ref[...]


def _update_pool(h, parts, Wn, bn, bcol, W1, b1, W2, b2):
    return pl.pallas_call(
        _upool_body,
        grid=(_P_NB,),
        in_specs=[
            pl.BlockSpec((_P_RB, D), lambda i: (i, 0)),
            pl.BlockSpec((1, _P_RB, D), lambda i: (0, i, 0)),
            pl.BlockSpec((1, _P_RB, D), lambda i: (1, i, 0)),
            pl.BlockSpec((D, D), lambda i: (0, 0)),
            pl.BlockSpec((1, D), lambda i: (0, 0)),
            pl.BlockSpec((_P_RB, 1), lambda i: (i, 0)),
            pl.BlockSpec((2 * D, D // 2), lambda i: (0, 0)),
            pl.BlockSpec((1, D // 2), lambda i: (0, 0)),
            pl.BlockSpec((D // 2, OUT), lambda i: (0, 0)),
            pl.BlockSpec((1, OUT), lambda i: (0, 0)),
        ],
        out_specs=pl.BlockSpec((G, OUT), lambda i: (0, 0)),
        out_shape=jax.ShapeDtypeStruct((G, OUT), jnp.float32),
        scratch_shapes=[pltpu.VMEM((G, D), jnp.float32),
                        pltpu.VMEM((G, D), jnp.float32),
                        pltpu.VMEM((G, 1), jnp.float32)],
    )(h, parts, parts, Wn, bn, bcol, W1, b1, W2, b2)


def kernel(x, edge_index, edge_attr, batch,
           We0, be0, Wn0, bn0, We1, be1, Wn1, bn1, We2, be2, Wn2, bn2,
           W1, b1, W2, b2):
    pad = E_PAD - E
    src = jnp.concatenate(
        [edge_index[0], jnp.arange(pad, dtype=jnp.int32) % N])
    dst = jnp.concatenate(
        [edge_index[1], N + (jnp.arange(pad, dtype=jnp.int32) % (ACC_ROWS - N))])
    srcp = src.reshape(E_PAD // EB, EB)
    dstp = dst.reshape(E_PAD // EB, EB)

    e0l, e0r = _edge_mlp1(edge_attr, We0, be0.reshape(1, D))
    parts = _sc_aggregate(x, e0l, e0r, srcp, dstp)
    e1l, e1r, e2l, e2r = _edge_mlp2(edge_attr, We1, be1.reshape(1, D),
                                    We2, be2.reshape(1, D))
    h = _update(x, parts, Wn0, bn0.reshape(1, D))

    parts = _sc_aggregate(h, e1l, e1r, srcp, dstp)
    h = _update(h, parts, Wn1, bn1.reshape(1, D))

    parts = _sc_aggregate(h, e2l, e2r, srcp, dstp)
    return _update_pool(h, parts, Wn2, bn2.reshape(1, D), batch.reshape(N, 1),
                        W1, b1.reshape(1, D // 2), W2, b2.reshape(1, OUT))
